# baseline (speedup 1.0000x reference)
import functools

import jax
import jax.numpy as jnp
from jax import lax
from jax.experimental import pallas as pl
from jax.experimental.pallas import tpu as pltpu
from jax.experimental.pallas import tpu_sc as plsc

_EPS = 1e-5


def _sim_block(x_ref, emb_ref, out_ref):
    x = x_ref[...]
    emb = emb_ref[...]
    m = jnp.mean(x, axis=1, keepdims=True)
    q = jnp.mean(x * x, axis=1, keepdims=True)
    v = q - m * m
    xn = ((x - m) * jax.lax.rsqrt(v + _EPS)).astype(jnp.bfloat16)
    out_ref[...] = jax.lax.dot_general(
        emb.astype(jnp.bfloat16), xn, (((1,), (1,)), ((), ())),
        preferred_element_type=jnp.float32,
    )


def _tc_sim(x, emb, n, d, e, block_t=2048):
    return pl.pallas_call(
        _sim_block,
        grid=(n // block_t,),
        in_specs=[
            pl.BlockSpec((block_t, d), lambda i: (i, 0)),
            pl.BlockSpec((e, d), lambda i: (0, 0)),
        ],
        out_specs=pl.BlockSpec((8, block_t), lambda i: (0, i)),
        out_shape=jax.ShapeDtypeStruct((8, n), jnp.float32),
    )(x, emb)


def _make_sc_router(n, temp):
    info = plsc.get_sparse_core_info()
    nc, ns = info.num_cores, info.num_subcores
    nw = nc * ns
    tok_w = n // nw
    chunks = tok_w // 16
    mesh = plsc.VectorSubcoreMesh(core_axis_name="c", subcore_axis_name="s")
    inv_temp = 1.0 / temp

    @functools.partial(
        pl.kernel, mesh=mesh,
        out_type=[jax.ShapeDtypeStruct((2 * n,), jnp.int32),
                  jax.ShapeDtypeStruct((2 * n,), jnp.float32)],
        scratch_types=[pltpu.VMEM((8, tok_w), jnp.float32),
                       pltpu.VMEM((2 * tok_w,), jnp.int32),
                       pltpu.VMEM((2 * tok_w,), jnp.float32)],
    )
    def sc_router(sim_hbm, idx_hbm, prob_hbm, sim_v, idx_v, prob_v):
        wid = lax.axis_index("s") * nc + lax.axis_index("c")
        base = wid * tok_w
        for r in range(8):
            pltpu.sync_copy(sim_hbm.at[r, pl.ds(base, tok_w)], sim_v.at[r])

        lane = lax.iota(jnp.int32, 16)
        pair_src = lane // 2
        even = (lane % 2) == 0

        def interleave(a, b):
            lo = jnp.where(even, a.at[pair_src].get(mode="promise_in_bounds"),
                           b.at[pair_src].get(mode="promise_in_bounds"))
            hi = jnp.where(even, a.at[pair_src + 8].get(mode="promise_in_bounds"),
                           b.at[pair_src + 8].get(mode="promise_in_bounds"))
            return lo, hi

        def body(c, carry):
            t = c * 16
            s = [sim_v[r, pl.ds(t, 16)] for r in range(8)]
            max1 = s[0]
            for r in range(1, 8):
                max1 = jnp.maximum(max1, s[r])
            idx1 = jnp.zeros((16,), jnp.int32)
            for r in range(7, -1, -1):
                idx1 = jnp.where(s[r] == max1, r, idx1)
            neg = jnp.float32(-jnp.inf)
            sm = [jnp.where(idx1 == r, neg, s[r]) for r in range(8)]
            max2 = sm[0]
            for r in range(1, 8):
                max2 = jnp.maximum(max2, sm[r])
            idx2 = jnp.zeros((16,), jnp.int32)
            for r in range(7, -1, -1):
                idx2 = jnp.where(sm[r] == max2, r, idx2)
            e2 = jnp.exp((max2 - max1) * inv_temp)
            denom = 1.0 + e2
            p1 = 1.0 / denom
            p2 = e2 / denom

            idx_v[pl.ds(t, 16)] = idx1
            idx_v[pl.ds(tok_w + t, 16)] = idx2
            prob_v[pl.ds(t, 16)] = p1
            prob_v[pl.ds(tok_w + t, 16)] = p2
            return carry

        lax.fori_loop(0, chunks, body, 0)
        pltpu.sync_copy(idx_v.at[pl.ds(0, tok_w)], idx_hbm.at[pl.ds(base, tok_w)])
        pltpu.sync_copy(idx_v.at[pl.ds(tok_w, tok_w)], idx_hbm.at[pl.ds(n + base, tok_w)])
        pltpu.sync_copy(prob_v.at[pl.ds(0, tok_w)], prob_hbm.at[pl.ds(base, tok_w)])
        pltpu.sync_copy(prob_v.at[pl.ds(tok_w, tok_w)], prob_hbm.at[pl.ds(n + base, tok_w)])

    return sc_router


def kernel(input, expert_embeddings):
    b, s, d = input.shape
    e = expert_embeddings.shape[0]
    n = b * s
    x = input.reshape(n, d)
    sim = _tc_sim(x, expert_embeddings, n, d, e)
    idx_flat, prob_flat = _make_sc_router(n, float(d) ** 0.5)(sim)
    idx = idx_flat.reshape(2, n).T.reshape(b, s, 2)
    probs = prob_flat.reshape(2, n).T.reshape(b, s, 2)
    return idx, probs


# speedup vs baseline: 1.4834x; 1.4834x over previous
import functools

import jax
import jax.numpy as jnp
from jax.experimental import pallas as pl

_EPS = 1e-5


def _router_block(x_ref, emb_ref, out_ref, *, temp):
    x = x_ref[...]
    emb = emb_ref[...]

    d = x.shape[1]
    acc_s = x[:, 0:128]
    acc_q = acc_s * acc_s
    for k in range(1, d // 128):
        c = x[:, 128 * k:128 * (k + 1)]
        acc_s = acc_s + c
        acc_q = acc_q + c * c
    inv_d = 1.0 / d
    m = jnp.sum(acc_s, axis=1, keepdims=True) * inv_d
    q = jnp.sum(acc_q, axis=1, keepdims=True) * inv_d
    v = q - m * m
    xn = ((x - m) * jax.lax.rsqrt(v + _EPS)).astype(jnp.bfloat16)

    sim = jax.lax.dot_general(
        emb.astype(jnp.bfloat16), xn, (((1,), (1,)), ((), ())),
        preferred_element_type=jnp.float32,
    )

    iota = jax.lax.broadcasted_iota(jnp.int32, sim.shape, 0)
    max1 = jnp.max(sim, axis=0, keepdims=True)
    idx1 = jnp.min(jnp.where(sim == max1, iota, 8), axis=0, keepdims=True)
    masked = jnp.where(iota == idx1, -jnp.inf, sim)
    max2 = jnp.max(masked, axis=0, keepdims=True)
    idx2 = jnp.min(jnp.where(masked == max2, iota, 8), axis=0, keepdims=True)

    e2 = jnp.exp((max2 - max1) / temp)
    denom = 1.0 + e2
    p1 = 1.0 / denom
    p2 = e2 / denom

    i1f = idx1.astype(jnp.float32)
    i2f = idx2.astype(jnp.float32)
    out_ref[...] = jnp.concatenate([i1f, i2f, p1, p2, i1f, i2f, p1, p2], axis=0)


def kernel(input, expert_embeddings):
    b, s, d = input.shape
    e = expert_embeddings.shape[0]
    n = b * s
    x = input.reshape(n, d)

    block_t = 2048
    grid = (n // block_t,)
    temp = float(d) ** 0.5

    out = pl.pallas_call(
        functools.partial(_router_block, temp=temp),
        grid=grid,
        in_specs=[
            pl.BlockSpec((block_t, d), lambda i: (i, 0)),
            pl.BlockSpec((e, d), lambda i: (0, 0)),
        ],
        out_specs=pl.BlockSpec((8, block_t), lambda i: (0, i)),
        out_shape=jax.ShapeDtypeStruct((8, n), jnp.float32),
    )(x, expert_embeddings)

    idx = out[0:2, :].astype(jnp.int32).T.reshape(b, s, 2)
    probs = out[2:4, :].T.reshape(b, s, 2)
    return idx, probs


# speedup vs baseline: 1.5686x; 1.0575x over previous
import functools

import jax
import jax.numpy as jnp
from jax.experimental import pallas as pl

_EPS = 1e-5


def _router_block(x_ref, emb_ref, out_ref, *, temp):
    x = x_ref[...]
    emb = emb_ref[...]

    d = x.shape[1]
    acc_s = x[:, 0:128]
    acc_q = acc_s * acc_s
    for k in range(1, d // 128):
        c = x[:, 128 * k:128 * (k + 1)]
        acc_s = acc_s + c
        acc_q = acc_q + c * c
    inv_d = 1.0 / d
    m = jnp.sum(acc_s, axis=1, keepdims=True) * inv_d
    q = jnp.sum(acc_q, axis=1, keepdims=True) * inv_d
    v = q - m * m
    xn = ((x - m) * jax.lax.rsqrt(v + _EPS)).astype(jnp.bfloat16)

    sim = jax.lax.dot_general(
        emb.astype(jnp.bfloat16), xn, (((1,), (1,)), ((), ())),
        preferred_element_type=jnp.float32,
    )

    iota = jax.lax.broadcasted_iota(jnp.int32, sim.shape, 0)
    max1 = jnp.max(sim, axis=0, keepdims=True)
    idx1 = jnp.min(jnp.where(sim == max1, iota, 8), axis=0, keepdims=True)
    masked = jnp.where(iota == idx1, -jnp.inf, sim)
    max2 = jnp.max(masked, axis=0, keepdims=True)
    idx2 = jnp.min(jnp.where(masked == max2, iota, 8), axis=0, keepdims=True)

    e2 = jnp.exp((max2 - max1) / temp)
    denom = 1.0 + e2
    p1 = 1.0 / denom
    p2 = e2 / denom

    i1f = idx1.astype(jnp.float32)
    i2f = idx2.astype(jnp.float32)
    out_ref[...] = jnp.concatenate([i1f, i2f, p1, p2, i1f, i2f, p1, p2], axis=0)


def kernel(input, expert_embeddings):
    b, s, d = input.shape
    e = expert_embeddings.shape[0]
    n = b * s
    x = input.reshape(n, d)

    block_t = 4096
    grid = (n // block_t,)
    temp = float(d) ** 0.5

    out = pl.pallas_call(
        functools.partial(_router_block, temp=temp),
        grid=grid,
        in_specs=[
            pl.BlockSpec((block_t, d), lambda i: (i, 0)),
            pl.BlockSpec((e, d), lambda i: (0, 0)),
        ],
        out_specs=pl.BlockSpec((8, block_t), lambda i: (0, i)),
        out_shape=jax.ShapeDtypeStruct((8, n), jnp.float32),
    )(x, expert_embeddings)

    idx = out[0:2, :].astype(jnp.int32).T.reshape(b, s, 2)
    probs = out[2:4, :].T.reshape(b, s, 2)
    return idx, probs
